# Initial kernel scaffold; baseline (speedup 1.0000x reference)
import functools

import jax
import jax.numpy as jnp
from jax import lax
from jax.experimental import pallas as pl
from jax.experimental.pallas import tpu as pltpu
from jax.experimental.pallas import tpu_sc as plsc

_CHUNK = 128
_NBUF = 8


@functools.lru_cache(maxsize=None)
def _make_gather(B, D):
    info = plsc.get_sparse_core_info()
    nc, ns = info.num_cores, info.num_subcores
    nw = nc * ns
    assert B % (nw * _CHUNK) == 0, (B, nw)
    nchunk = B // (nw * _CHUNK)
    assert nchunk % _NBUF == 0 and nchunk > _NBUF
    rows_per_w = nchunk * _CHUNK
    mesh = plsc.VectorSubcoreMesh(core_axis_name="c", subcore_axis_name="s")

    @functools.partial(
        pl.kernel,
        out_type=jax.ShapeDtypeStruct((B, D), jnp.float32),
        mesh=mesh,
        scratch_types=[
            pltpu.VMEM((nchunk, _CHUNK), jnp.int32),
            pltpu.VMEM((_NBUF, _CHUNK, D), jnp.float32),
            pltpu.SemaphoreType.DMA((_NBUF,)),
            pltpu.SemaphoreType.DMA((_NBUF,)),
        ],
    )
    def gather_kernel(idx_hbm, table_hbm, out_hbm, idx_v, rows_v, gsem, osem):
        wid = lax.axis_index("s") * nc + lax.axis_index("c")
        base = wid * rows_per_w
        pltpu.sync_copy(idx_hbm.at[wid], idx_v)

        def gather_desc(j, b):
            return pltpu.make_async_copy(
                table_hbm.at[idx_v.at[j]], rows_v.at[b], gsem.at[b])

        def ostore_desc(j, b):
            return pltpu.make_async_copy(
                rows_v.at[b], out_hbm.at[pl.ds(base + j * _CHUNK, _CHUNK)],
                osem.at[b])

        for b in range(_NBUF):
            gather_desc(b, b).start()

        def round_body(r, carry):
            j0 = r * _NBUF
            for b in range(_NBUF):
                gather_desc(j0 + b, b).wait()
                ostore_desc(j0 + b, b).start()
            for b in range(_NBUF):
                ostore_desc(j0 + b, b).wait()
                gather_desc(j0 + _NBUF + b, b).start()
            return carry

        lax.fori_loop(0, nchunk // _NBUF - 1, round_body, 0)

        j0 = nchunk - _NBUF
        for b in range(_NBUF):
            gather_desc(j0 + b, b).wait()
            ostore_desc(j0 + b, b).start()
        for b in range(_NBUF):
            ostore_desc(j0 + b, b).wait()

    return gather_kernel, nw, nchunk


def kernel(x, table):
    n, s = x.shape
    _, d = table.shape
    b = n * s
    fn, nw, nchunk = _make_gather(b, d)
    idx = x.astype(jnp.int32).reshape(nw, nchunk, _CHUNK)
    out = fn(idx, table)
    return out.reshape(n, s, d)


# speedup vs baseline: 1.8708x; 1.8708x over previous
import functools

import jax
import jax.numpy as jnp
from jax import lax
from jax.experimental import pallas as pl
from jax.experimental.pallas import tpu as pltpu
from jax.experimental.pallas import tpu_sc as plsc

_CHUNK = 128
_NBUF = 8


@functools.lru_cache(maxsize=None)
def _make_gather(B, D):
    info = plsc.get_sparse_core_info()
    nc, ns = info.num_cores, info.num_subcores
    nw = nc * ns
    assert B % (nw * _CHUNK) == 0, (B, nw)
    nchunk = B // (nw * _CHUNK)
    assert nchunk % _NBUF == 0 and nchunk > _NBUF
    rows_per_w = nchunk * _CHUNK
    mesh = plsc.VectorSubcoreMesh(core_axis_name="c", subcore_axis_name="s")

    @functools.partial(
        pl.kernel,
        out_type=jax.ShapeDtypeStruct((B, D), jnp.float32),
        mesh=mesh,
        compiler_params=pltpu.CompilerParams(use_tc_tiling_on_sc=False),
        scratch_types=[
            pltpu.VMEM((nchunk, _CHUNK), jnp.int32),
            pltpu.VMEM((_NBUF, _CHUNK, D), jnp.float32),
            pltpu.SemaphoreType.DMA((_NBUF,)),
            pltpu.SemaphoreType.DMA((_NBUF,)),
        ],
    )
    def gather_kernel(idx_hbm, table_hbm, out_hbm, idx_v, rows_v, gsem, osem):
        wid = lax.axis_index("s") * nc + lax.axis_index("c")
        base = wid * rows_per_w
        pltpu.sync_copy(idx_hbm.at[wid], idx_v)

        def gather_desc(j, b):
            return pltpu.make_async_copy(
                table_hbm.at[idx_v.at[j]], rows_v.at[b], gsem.at[b])

        def ostore_desc(j, b):
            return pltpu.make_async_copy(
                rows_v.at[b], out_hbm.at[pl.ds(base + j * _CHUNK, _CHUNK)],
                osem.at[b])

        for b in range(_NBUF):
            gather_desc(b, b).start()

        def round_body(r, carry):
            j0 = r * _NBUF
            for b in range(_NBUF):
                gather_desc(j0 + b, b).wait()
                ostore_desc(j0 + b, b).start()
            for b in range(_NBUF):
                ostore_desc(j0 + b, b).wait()
                gather_desc(j0 + _NBUF + b, b).start()
            return carry

        lax.fori_loop(0, nchunk // _NBUF - 1, round_body, 0)

        j0 = nchunk - _NBUF
        for b in range(_NBUF):
            gather_desc(j0 + b, b).wait()
            ostore_desc(j0 + b, b).start()
        for b in range(_NBUF):
            ostore_desc(j0 + b, b).wait()

    return gather_kernel, nw, nchunk


def kernel(x, table):
    n, s = x.shape
    _, d = table.shape
    b = n * s
    fn, nw, nchunk = _make_gather(b, d)
    idx = x.astype(jnp.int32).reshape(nw, nchunk, _CHUNK)
    out = fn(idx, table)
    return out.reshape(n, s, d)
